# baseline (speedup 1.0000x reference)
import functools

import jax
import jax.numpy as jnp
import numpy as np
from jax import lax
from jax.experimental import pallas as pl
from jax.experimental.pallas import tpu as pltpu
from jax.experimental.pallas import tpu_sc as plsc

NUM_LEVELS = 16
TABLE_SIZE = 524288
FEATURE_DIM = 2
MIN_RES = 16
MAX_RES = 2048
N_POINTS = 262144
FEAT = NUM_LEVELS * FEATURE_DIM
NROW = NUM_LEVELS * TABLE_SIZE

_b = np.exp((np.log(MAX_RES) - np.log(MIN_RES)) / (NUM_LEVELS - 1))
_RES = [int(np.floor(MIN_RES * (_b**l))) for l in range(NUM_LEVELS)]
_HASH_MUL = int(np.uint32(2654435761).view(np.int32))
_MASK = TABLE_SIZE - 1
_CORNERS = ((0, 0), (0, 1), (1, 0), (1, 1))

NC, NS = 2, 16
NW = NC * NS
LANES = 16
C = 64

N_RES_LVL = 7
_ROWS_L = [r * (r + 2) + 1 for r in _RES]
_SL_L = [(rw + 3) // 4 for rw in _ROWS_L]
RES_OFF = [0]
for _l in range(N_RES_LVL):
    RES_OFF.append(RES_OFF[-1] + _SL_L[_l])
RES_SLICES = RES_OFF[-1]
NSTREAMED = (NUM_LEVELS - N_RES_LVL) * 4
XB = 128
PPW = N_POINTS // NW
NCHUNK = PPW // C
NCHUNK_G = N_POINTS // C
NK = NUM_LEVELS * 4


QB = 1024


def _retile_body(t_ref, o_ref):
    z = t_ref[...]
    a = z[0, :, 0, :]
    b = z[0, :, 1, :]
    r64 = lax.broadcasted_iota(jnp.int32, (64, 128), 0)
    c128 = lax.broadcasted_iota(jnp.int32, (64, 128), 1)
    ev = (c128 == 2 * r64).astype(jnp.float32)
    od = (c128 == 2 * r64 + 1).astype(jnp.float32)
    outs = []
    for h_i in (0, 1):
        ah = a[:, 64 * h_i:64 * h_i + 64].astype(jnp.bfloat16)
        bh = b[:, 64 * h_i:64 * h_i + 64].astype(jnp.bfloat16)
        outs.append(
            jnp.dot(ah, ev.astype(jnp.bfloat16),
                    preferred_element_type=jnp.float32)
            + jnp.dot(bh, od.astype(jnp.bfloat16),
                      preferred_element_type=jnp.float32))
    o_ref[...] = jnp.stack(outs, axis=1).reshape(2 * QB, 128)


def _retile(tq):
    return pl.pallas_call(
        _retile_body,
        grid=(NUM_LEVELS, TABLE_SIZE // XB // QB),
        in_specs=[pl.BlockSpec((1, QB, 2, 128), lambda l, q: (l, q, 0, 0))],
        out_specs=pl.BlockSpec((2 * QB, 128),
                               lambda l, q: (l * (TABLE_SIZE // XB // QB) + q,
                                             0)),
        out_shape=jax.ShapeDtypeStruct((NROW * 2 // 128, 128), jnp.float32),
    )(tq)


def _enc_body(xT_hbm, tab_hbm, h_hbm, xbuf, wxbuf, wybuf, idxbuf, idx4buf,
              rowsbuf, resbuf, hbuf, sem0, sem1):
    wid = lax.axis_index("s") * NC + lax.axis_index("c")
    iota = lax.iota(jnp.int32, LANES)
    one16 = jnp.ones((LANES,), jnp.int32)
    sems = (sem0, sem1)

    for l in range(N_RES_LVL):
        pltpu.sync_copy(
            tab_hbm.at[pl.ds(l * (TABLE_SIZE // 4), _SL_L[l])],
            resbuf.at[pl.ds(RES_OFF[l], _SL_L[l])])

    def produce(ci, slot):
        g = wid * NCHUNK + ci
        xoff = (g // 2) * (2 * XB) + (g % 2) * C
        pltpu.sync_copy(xT_hbm.at[pl.ds(xoff, C)], xbuf.at[slot, 0])
        pltpu.sync_copy(xT_hbm.at[pl.ds(xoff + XB, C)], xbuf.at[slot, 1])

        def slice_a(s, carry_a):
            p0 = s * LANES
            xv = xbuf[slot, 0, pl.ds(p0, LANES)]
            yv = xbuf[slot, 1, pl.ds(p0, LANES)]
            for l in range(NUM_LEVELS):
                res = _RES[l]
                xs = xv * jnp.float32(res)
                ys = yv * jnp.float32(res)
                x0 = xs.astype(jnp.int32)
                y0 = ys.astype(jnp.int32)
                wxbuf[slot, l, pl.ds(p0, LANES)] = xs - x0.astype(jnp.float32)
                wybuf[slot, l, pl.ds(p0, LANES)] = ys - y0.astype(jnp.float32)
                if (res + 1) ** 2 <= TABLE_SIZE:
                    base00 = x0 + y0 * jnp.int32(res + 1)
                    corner_idx = (base00, base00 + jnp.int32(res + 1),
                                  base00 + 1, base00 + jnp.int32(res + 2))
                else:
                    m0 = y0 * jnp.int32(_HASH_MUL)
                    m1 = m0 + jnp.int32(_HASH_MUL)
                    x1 = x0 + 1
                    msk = jnp.int32(_MASK)
                    corner_idx = ((x0 ^ m0) & msk, (x0 ^ m1) & msk,
                                  (x1 ^ m0) & msk, (x1 ^ m1) & msk)
                for c_i in range(4):
                    idx = corner_idx[c_i]
                    if l < N_RES_LVL:
                        idxbuf[slot, 4 * l + c_i, pl.ds(p0, LANES)] = idx
                    else:
                        gi = idx + jnp.int32(l * TABLE_SIZE)
                        idxbuf[slot, 4 * l + c_i, pl.ds(p0, LANES)] = gi
                        idx4buf[slot, 4 * (l - N_RES_LVL) + c_i,
                                pl.ds(p0, LANES)] = gi >> 2
            return carry_a

        lax.fori_loop(0, C // LANES, slice_a, 0)
        for k in range(NSTREAMED):
            pltpu.async_copy(tab_hbm.at[idx4buf.at[slot, k]],
                             rowsbuf.at[slot, k], sems[slot])

    def consume(ci, slot):
        g = wid * NCHUNK + ci
        for k in range(NSTREAMED):
            pltpu.make_async_copy(tab_hbm.at[idx4buf.at[slot, k]],
                                  rowsbuf.at[slot, k], sems[slot]).wait()

        def slice_c(s, carry_c):
            p0 = s * LANES
            pvec = iota + p0
            for l in range(NUM_LEVELS):
                wx = wxbuf[slot, l, pl.ds(p0, LANES)]
                wy = wybuf[slot, l, pl.ds(p0, LANES)]
                omx = 1.0 - wx
                omy = 1.0 - wy
                wc = (omx * omy, omx * wy, wx * omy, wx * wy)
                acc0 = jnp.zeros((LANES,), jnp.float32)
                acc1 = jnp.zeros((LANES,), jnp.float32)
                for c_i in range(4):
                    iv = idxbuf[slot, 4 * l + c_i, pl.ds(p0, LANES)]
                    sub = (iv & 3) * 2
                    if l < N_RES_LVL:
                        svec = (iv >> 2) + jnp.int32(RES_OFF[l])
                        f0 = plsc.load_gather(resbuf, [svec, sub])
                        f1 = plsc.load_gather(resbuf, [svec, sub + one16])
                    else:
                        kvec = jnp.full((LANES,),
                                        4 * (l - N_RES_LVL) + c_i, jnp.int32)
                        f0 = plsc.load_gather(rowsbuf.at[slot],
                                              [kvec, pvec, sub])
                        f1 = plsc.load_gather(rowsbuf.at[slot],
                                              [kvec, pvec, sub + one16])
                    acc0 = acc0 + wc[c_i] * f0
                    acc1 = acc1 + wc[c_i] * f1
                hbuf[2 * l, pl.ds(p0, LANES)] = acc0
                hbuf[2 * l + 1, pl.ds(p0, LANES)] = acc1
            return carry_c

        lax.fori_loop(0, C // LANES, slice_c, 0)
        pltpu.sync_copy(hbuf, h_hbm.at[g])

    def pair_body(j, carry):
        produce(2 * j, 0)

        @pl.when(j > 0)
        def _():
            consume(2 * j - 1, 1)

        produce(2 * j + 1, 1)
        consume(2 * j, 0)
        return carry

    lax.fori_loop(0, NCHUNK // 2, pair_body, 0)
    consume(NCHUNK - 1, 1)


_encode = functools.partial(
    pl.kernel,
    out_type=jax.ShapeDtypeStruct((NCHUNK_G, FEAT, C), jnp.float32),
    mesh=plsc.VectorSubcoreMesh(
        core_axis_name="c", subcore_axis_name="s", num_cores=NC,
        num_subcores=NS),
    compiler_params=pltpu.CompilerParams(
        needs_layout_passes=False, use_tc_tiling_on_sc=False),
    scratch_types=[
        pltpu.VMEM((2, 2, C), jnp.float32),
        pltpu.VMEM((2, NUM_LEVELS, C), jnp.float32),
        pltpu.VMEM((2, NUM_LEVELS, C), jnp.float32),
        pltpu.VMEM((2, NK, C), jnp.int32),
        pltpu.VMEM((2, NSTREAMED, C), jnp.int32),
        pltpu.VMEM((2, NSTREAMED, C, 8), jnp.float32),
        pltpu.VMEM((RES_SLICES, 8), jnp.float32),
        pltpu.VMEM((FEAT, C), jnp.float32),
        pltpu.SemaphoreType.DMA,
        pltpu.SemaphoreType.DMA,
    ],
)(_enc_body)


BLK_CHUNKS = 32
BLK = BLK_CHUNKS * C


def _mlp_body(h_ref, w1_ref, b1_ref, w2_ref, b2_ref, w3_ref, b3_ref, wo_ref,
              bo_ref, o_ref):
    hb = h_ref[...]
    z = jnp.transpose(hb, (0, 2, 1)).reshape(BLK, FEAT)

    def mm(zz, w_ref):
        return jnp.dot(zz.astype(jnp.bfloat16),
                       w_ref[...].astype(jnp.bfloat16),
                       preferred_element_type=jnp.float32)

    z = jnp.maximum(mm(z, w1_ref) + b1_ref[...], 0.0)
    z = jnp.maximum(mm(z, w2_ref) + b2_ref[...], 0.0)
    z = jnp.maximum(mm(z, w3_ref) + b3_ref[...], 0.0)
    z = mm(z, wo_ref) + bo_ref[...]
    z = jax.nn.sigmoid(z)
    o_ref[...] = jnp.transpose(z.reshape(BLK // XB, XB, 4), (0, 2, 1))


def _full_spec(shape):
    return pl.BlockSpec(shape, lambda i, _s=shape: tuple(0 for _ in _s))


def _mlp(h, W1, b1, W2, b2, W3, b3, Wout, bout):
    return pl.pallas_call(
        _mlp_body,
        grid=(N_POINTS // BLK,),
        in_specs=[
            pl.BlockSpec((BLK_CHUNKS, FEAT, C), lambda i: (i, 0, 0)),
            _full_spec(W1.shape),
            _full_spec(b1.shape),
            _full_spec(W2.shape),
            _full_spec(b2.shape),
            _full_spec(W3.shape),
            _full_spec(b3.shape),
            _full_spec(Wout.shape),
            _full_spec(bout.shape),
        ],
        out_specs=pl.BlockSpec((BLK // XB, 4, XB), lambda i: (i, 0, 0)),
        out_shape=jax.ShapeDtypeStruct((N_POINTS // XB, 4, XB), jnp.float32),
    )(h, W1, b1, W2, b2, W3, b3, Wout, bout)


def kernel(x, table, W1, b1, W2, b2, W3, b3, Wout, bout):
    xP = x.reshape(N_POINTS // XB, XB, 2).transpose(0, 2, 1).reshape(
        N_POINTS * 2)
    tq = table.reshape(NUM_LEVELS, TABLE_SIZE // XB, XB,
                       FEATURE_DIM).transpose(0, 1, 3, 2)
    tab = _retile(tq).reshape(NROW // 4, 8)
    h = _encode(xP, tab)
    y = _mlp(h, W1, b1.reshape(1, -1), W2, b2.reshape(1, -1), W3,
             b3.reshape(1, -1), Wout, bout.reshape(1, -1))
    return y.transpose(0, 2, 1).reshape(N_POINTS, 4)


# speedup vs baseline: 1.0083x; 1.0083x over previous
import functools

import jax
import jax.numpy as jnp
import numpy as np
from jax import lax
from jax.experimental import pallas as pl
from jax.experimental.pallas import tpu as pltpu
from jax.experimental.pallas import tpu_sc as plsc

NUM_LEVELS = 16
TABLE_SIZE = 524288
FEATURE_DIM = 2
MIN_RES = 16
MAX_RES = 2048
N_POINTS = 262144
FEAT = NUM_LEVELS * FEATURE_DIM
NROW = NUM_LEVELS * TABLE_SIZE

_b = np.exp((np.log(MAX_RES) - np.log(MIN_RES)) / (NUM_LEVELS - 1))
_RES = [int(np.floor(MIN_RES * (_b**l))) for l in range(NUM_LEVELS)]
_HASH_MUL = int(np.uint32(2654435761).view(np.int32))
_MASK = TABLE_SIZE - 1
_CORNERS = ((0, 0), (0, 1), (1, 0), (1, 1))

NC, NS = 2, 16
NW = NC * NS
LANES = 16
C = 64

N_RES_LVL = 7
_ROWS_L = [r * (r + 2) + 1 for r in _RES]
_SL_L = [(rw + 3) // 4 for rw in _ROWS_L]
RES_OFF = [0]
for _l in range(N_RES_LVL):
    RES_OFF.append(RES_OFF[-1] + _SL_L[_l])
RES_SLICES = RES_OFF[-1]
NSTREAMED = (NUM_LEVELS - N_RES_LVL) * 4
XB = 128
PPW = N_POINTS // NW
NCHUNK = PPW // C
NCHUNK_G = N_POINTS // C
NK = NUM_LEVELS * 4


QB = 1024


def _retile_body(t_ref, o_ref):
    z = t_ref[...]
    a = z[0, :, 0, :]
    b = z[0, :, 1, :]
    r64 = lax.broadcasted_iota(jnp.int32, (64, 128), 0)
    c128 = lax.broadcasted_iota(jnp.int32, (64, 128), 1)
    ev = (c128 == 2 * r64).astype(jnp.float32)
    od = (c128 == 2 * r64 + 1).astype(jnp.float32)
    outs = []
    for h_i in (0, 1):
        ah = a[:, 64 * h_i:64 * h_i + 64]
        bh = b[:, 64 * h_i:64 * h_i + 64]
        outs.append(
            jnp.dot(ah, ev, preferred_element_type=jnp.float32)
            + jnp.dot(bh, od, preferred_element_type=jnp.float32))
    o_ref[...] = jnp.stack(outs, axis=1).reshape(2 * QB, 128)


def _retile(tq):
    return pl.pallas_call(
        _retile_body,
        grid=(NUM_LEVELS, TABLE_SIZE // XB // QB),
        in_specs=[pl.BlockSpec((1, QB, 2, 128), lambda l, q: (l, q, 0, 0))],
        out_specs=pl.BlockSpec((2 * QB, 128),
                               lambda l, q: (l * (TABLE_SIZE // XB // QB) + q,
                                             0)),
        out_shape=jax.ShapeDtypeStruct((NROW * 2 // 128, 128), jnp.float32),
    )(tq)


def _enc_body(xT_hbm, tab_hbm, h_hbm, xbuf, wxbuf, wybuf, idxbuf, idx4buf,
              rowsbuf, resbuf, hbuf, sem0, sem1):
    wid = lax.axis_index("s") * NC + lax.axis_index("c")
    iota = lax.iota(jnp.int32, LANES)
    one16 = jnp.ones((LANES,), jnp.int32)
    sems = (sem0, sem1)

    for l in range(N_RES_LVL):
        pltpu.sync_copy(
            tab_hbm.at[pl.ds(l * (TABLE_SIZE // 4), _SL_L[l])],
            resbuf.at[pl.ds(RES_OFF[l], _SL_L[l])])

    def produce(ci, slot):
        g = wid * NCHUNK + ci
        xoff = (g // 2) * (2 * XB) + (g % 2) * C
        pltpu.sync_copy(xT_hbm.at[pl.ds(xoff, C)], xbuf.at[slot, 0])
        pltpu.sync_copy(xT_hbm.at[pl.ds(xoff + XB, C)], xbuf.at[slot, 1])

        def slice_a(s, carry_a):
            p0 = s * LANES
            xv = xbuf[slot, 0, pl.ds(p0, LANES)]
            yv = xbuf[slot, 1, pl.ds(p0, LANES)]
            for l in range(NUM_LEVELS):
                res = _RES[l]
                xs = xv * jnp.float32(res)
                ys = yv * jnp.float32(res)
                x0 = xs.astype(jnp.int32)
                y0 = ys.astype(jnp.int32)
                wxbuf[slot, l, pl.ds(p0, LANES)] = xs - x0.astype(jnp.float32)
                wybuf[slot, l, pl.ds(p0, LANES)] = ys - y0.astype(jnp.float32)
                if (res + 1) ** 2 <= TABLE_SIZE:
                    base00 = x0 + y0 * jnp.int32(res + 1)
                    corner_idx = (base00, base00 + jnp.int32(res + 1),
                                  base00 + 1, base00 + jnp.int32(res + 2))
                else:
                    m0 = y0 * jnp.int32(_HASH_MUL)
                    m1 = m0 + jnp.int32(_HASH_MUL)
                    x1 = x0 + 1
                    msk = jnp.int32(_MASK)
                    corner_idx = ((x0 ^ m0) & msk, (x0 ^ m1) & msk,
                                  (x1 ^ m0) & msk, (x1 ^ m1) & msk)
                for c_i in range(4):
                    idx = corner_idx[c_i]
                    if l < N_RES_LVL:
                        idxbuf[slot, 4 * l + c_i, pl.ds(p0, LANES)] = idx
                    else:
                        gi = idx + jnp.int32(l * TABLE_SIZE)
                        idxbuf[slot, 4 * l + c_i, pl.ds(p0, LANES)] = gi
                        idx4buf[slot, 4 * (l - N_RES_LVL) + c_i,
                                pl.ds(p0, LANES)] = gi >> 2
            return carry_a

        lax.fori_loop(0, C // LANES, slice_a, 0)
        for k in range(NSTREAMED):
            pltpu.async_copy(tab_hbm.at[idx4buf.at[slot, k]],
                             rowsbuf.at[slot, k], sems[slot])

    def consume(ci, slot):
        g = wid * NCHUNK + ci
        for k in range(NSTREAMED):
            pltpu.make_async_copy(tab_hbm.at[idx4buf.at[slot, k]],
                                  rowsbuf.at[slot, k], sems[slot]).wait()

        def slice_c(s, carry_c):
            p0 = s * LANES
            pvec = iota + p0
            for l in range(NUM_LEVELS):
                wx = wxbuf[slot, l, pl.ds(p0, LANES)]
                wy = wybuf[slot, l, pl.ds(p0, LANES)]
                omx = 1.0 - wx
                omy = 1.0 - wy
                wc = (omx * omy, omx * wy, wx * omy, wx * wy)
                acc0 = jnp.zeros((LANES,), jnp.float32)
                acc1 = jnp.zeros((LANES,), jnp.float32)
                for c_i in range(4):
                    iv = idxbuf[slot, 4 * l + c_i, pl.ds(p0, LANES)]
                    sub = (iv & 3) * 2
                    if l < N_RES_LVL:
                        svec = (iv >> 2) + jnp.int32(RES_OFF[l])
                        f0 = plsc.load_gather(resbuf, [svec, sub])
                        f1 = plsc.load_gather(resbuf, [svec, sub + one16])
                    else:
                        kvec = jnp.full((LANES,),
                                        4 * (l - N_RES_LVL) + c_i, jnp.int32)
                        f0 = plsc.load_gather(rowsbuf.at[slot],
                                              [kvec, pvec, sub])
                        f1 = plsc.load_gather(rowsbuf.at[slot],
                                              [kvec, pvec, sub + one16])
                    acc0 = acc0 + wc[c_i] * f0
                    acc1 = acc1 + wc[c_i] * f1
                hbuf[2 * l, pl.ds(p0, LANES)] = acc0
                hbuf[2 * l + 1, pl.ds(p0, LANES)] = acc1
            return carry_c

        lax.fori_loop(0, C // LANES, slice_c, 0)
        pltpu.sync_copy(hbuf, h_hbm.at[g])

    def pair_body(j, carry):
        produce(2 * j, 0)

        @pl.when(j > 0)
        def _():
            consume(2 * j - 1, 1)

        produce(2 * j + 1, 1)
        consume(2 * j, 0)
        return carry

    lax.fori_loop(0, NCHUNK // 2, pair_body, 0)
    consume(NCHUNK - 1, 1)


_encode = functools.partial(
    pl.kernel,
    out_type=jax.ShapeDtypeStruct((NCHUNK_G, FEAT, C), jnp.float32),
    mesh=plsc.VectorSubcoreMesh(
        core_axis_name="c", subcore_axis_name="s", num_cores=NC,
        num_subcores=NS),
    compiler_params=pltpu.CompilerParams(
        needs_layout_passes=False, use_tc_tiling_on_sc=False),
    scratch_types=[
        pltpu.VMEM((2, 2, C), jnp.float32),
        pltpu.VMEM((2, NUM_LEVELS, C), jnp.float32),
        pltpu.VMEM((2, NUM_LEVELS, C), jnp.float32),
        pltpu.VMEM((2, NK, C), jnp.int32),
        pltpu.VMEM((2, NSTREAMED, C), jnp.int32),
        pltpu.VMEM((2, NSTREAMED, C, 8), jnp.float32),
        pltpu.VMEM((RES_SLICES, 8), jnp.float32),
        pltpu.VMEM((FEAT, C), jnp.float32),
        pltpu.SemaphoreType.DMA,
        pltpu.SemaphoreType.DMA,
    ],
)(_enc_body)


BLK_CHUNKS = 32
BLK = BLK_CHUNKS * C


def _mlp_body(h_ref, w1_ref, b1_ref, w2_ref, b2_ref, w3_ref, b3_ref, wo_ref,
              bo_ref, o_ref):
    hb = h_ref[...]
    z = jnp.transpose(hb, (0, 2, 1)).reshape(BLK, FEAT)
    z = jnp.maximum(
        jnp.dot(z, w1_ref[...], preferred_element_type=jnp.float32)
        + b1_ref[...], 0.0)
    z = jnp.maximum(
        jnp.dot(z, w2_ref[...], preferred_element_type=jnp.float32)
        + b2_ref[...], 0.0)
    z = jnp.maximum(
        jnp.dot(z, w3_ref[...], preferred_element_type=jnp.float32)
        + b3_ref[...], 0.0)
    z = jnp.dot(z, wo_ref[...], preferred_element_type=jnp.float32) + bo_ref[...]
    z = jax.nn.sigmoid(z)
    o_ref[...] = jnp.transpose(z.reshape(BLK // XB, XB, 4), (0, 2, 1))


def _full_spec(shape):
    return pl.BlockSpec(shape, lambda i, _s=shape: tuple(0 for _ in _s))


def _mlp(h, W1, b1, W2, b2, W3, b3, Wout, bout):
    return pl.pallas_call(
        _mlp_body,
        grid=(N_POINTS // BLK,),
        in_specs=[
            pl.BlockSpec((BLK_CHUNKS, FEAT, C), lambda i: (i, 0, 0)),
            _full_spec(W1.shape),
            _full_spec(b1.shape),
            _full_spec(W2.shape),
            _full_spec(b2.shape),
            _full_spec(W3.shape),
            _full_spec(b3.shape),
            _full_spec(Wout.shape),
            _full_spec(bout.shape),
        ],
        out_specs=pl.BlockSpec((BLK // XB, 4, XB), lambda i: (i, 0, 0)),
        out_shape=jax.ShapeDtypeStruct((N_POINTS // XB, 4, XB), jnp.float32),
    )(h, W1, b1, W2, b2, W3, b3, Wout, bout)


def kernel(x, table, W1, b1, W2, b2, W3, b3, Wout, bout):
    xP = x.reshape(N_POINTS // XB, XB, 2).transpose(0, 2, 1).reshape(
        N_POINTS * 2)
    tq = table.reshape(NUM_LEVELS, TABLE_SIZE // XB, XB,
                       FEATURE_DIM).transpose(0, 1, 3, 2)
    tab = _retile(tq).reshape(NROW // 4, 8)
    h = _encode(xP, tab)
    y = _mlp(h, W1, b1.reshape(1, -1), W2, b2.reshape(1, -1), W3,
             b3.reshape(1, -1), Wout, bout.reshape(1, -1))
    return y.transpose(0, 2, 1).reshape(N_POINTS, 4)
